# baseline (speedup 1.0000x reference)
import functools

import jax
import jax.numpy as jnp
from jax import lax
from jax.experimental import pallas as pl
from jax.experimental.pallas import tpu as pltpu
from jax.experimental.pallas import tpu_sc as plsc

NUM_EXPERTS = 8
D = 768
D_FF = 1536
TOKENS = 2048
TILE = 128
SLOTS = 2 * TOKENS
NROWS = SLOTS + NUM_EXPERTS * TILE
NBLK = NROWS // TILE
NW = 32
TPW = TOKENS // NW


def _mm(a, b):
    return jnp.dot(a.astype(jnp.bfloat16), b.astype(jnp.bfloat16),
                   preferred_element_type=jnp.float32)


def _gate_route_body(x_ref, wg1_ref, bg1_ref, wg2_ref, bg2_ref, wg3_ref,
                     bg3_ref, wd_ref, bd_ref,
                     pos1_ref, pos2_ref, wr1_ref, wr2_ref, be_ref):
    x = x_ref[...]
    g = jax.nn.relu(_mm(x, wg1_ref[...]) + bg1_ref[...])
    g = jax.nn.relu(_mm(g, wg2_ref[...]) + bg2_ref[...])
    logits = _mm(g, wg3_ref[...]) + bg3_ref[...]
    logits = logits + (_mm(x, wd_ref[...]) + bd_ref[...]) * 0.1
    p = jax.nn.softmax(logits, axis=-1)

    lane = lax.broadcasted_iota(jnp.int32, (TOKENS, NUM_EXPERTS), 1)
    m1 = jnp.max(p, axis=-1, keepdims=True)
    i1 = jnp.min(jnp.where(p == m1, lane, NUM_EXPERTS), axis=-1, keepdims=True)
    pm = jnp.where(lane == i1, -jnp.inf, p)
    m2 = jnp.max(pm, axis=-1, keepdims=True)
    i2 = jnp.min(jnp.where(pm == m2, lane, NUM_EXPERTS), axis=-1, keepdims=True)
    e2 = jnp.exp(m2 - m1)
    s1 = 1.0 / (1.0 + e2)
    s2 = e2 / (1.0 + e2)
    s1 = jnp.where(s1 > 0.01, s1, 0.0)
    s2 = jnp.where(s2 > 0.01, s2, 0.0)

    oh1 = (lane == i1).astype(jnp.bfloat16)
    oh2 = (lane == i2).astype(jnp.bfloat16)
    oh = jnp.concatenate([oh1, oh2], axis=1)
    r256 = lax.broadcasted_iota(jnp.int32, (256, 256), 0)
    c256 = lax.broadcasted_iota(jnp.int32, (256, 256), 1)
    tri = (r256 > c256).astype(jnp.bfloat16)
    carry = jnp.zeros((1, 2 * NUM_EXPERTS), jnp.float32)
    parts = []
    for b in range(TOKENS // 256):
        blk = lax.slice(oh, (b * 256, 0), ((b + 1) * 256, 2 * NUM_EXPERTS))
        parts.append(jnp.dot(tri, blk, preferred_element_type=jnp.float32)
                     + carry)
        carry = carry + jnp.sum(blk.astype(jnp.float32), axis=0,
                                keepdims=True)
    pfx = jnp.concatenate(parts, axis=0)
    p1 = pfx[:, :NUM_EXPERTS]
    p2 = pfx[:, NUM_EXPERTS:]
    c1 = carry[:, :NUM_EXPERTS]
    c2 = carry[:, NUM_EXPERTS:]
    counts = (c1 + c2).astype(jnp.int32)
    padded = ((counts + (TILE - 1)) // TILE) * TILE
    r8 = lax.broadcasted_iota(jnp.int32, (NUM_EXPERTS, NUM_EXPERTS), 0)
    c8 = lax.broadcasted_iota(jnp.int32, (NUM_EXPERTS, NUM_EXPERTS), 1)
    triu8 = (r8 < c8).astype(jnp.bfloat16)
    off = jnp.dot(padded.astype(jnp.bfloat16), triu8,
                  preferred_element_type=jnp.float32)

    rank1 = jnp.sum(jnp.where(lane == i1, p1, 0.0), axis=1, keepdims=True)
    rank2 = jnp.sum(jnp.where(lane == i2, p2 + c1, 0.0), axis=1,
                    keepdims=True)
    off1 = jnp.sum(jnp.where(lane == i1, off, 0.0), axis=1, keepdims=True)
    off2 = jnp.sum(jnp.where(lane == i2, off, 0.0), axis=1, keepdims=True)
    pos1_ref[...] = (off1 + rank1).astype(jnp.int32)
    pos2_ref[...] = (off2 + rank2).astype(jnp.int32)

    ones128 = jnp.ones((1, 128), jnp.float32)
    wr1_ref[...] = s1 * ones128
    wr2_ref[...] = s2 * ones128

    total = jnp.sum(padded, axis=1, keepdims=True)
    bid = lax.broadcasted_iota(jnp.int32, (NBLK, 1), 0) * TILE
    ge = (bid >= off.astype(jnp.int32)).astype(jnp.int32)
    be = jnp.sum(ge, axis=1, keepdims=True) - 1
    be_ref[...] = jnp.where(bid < total, be, -1)


def _gate_route(x, Wg1, bg1, Wg2, bg2, Wg3, bg3, Wd, bd, interpret=False):
    return pl.pallas_call(
        _gate_route_body,
        out_shape=(
            jax.ShapeDtypeStruct((TOKENS, 1), jnp.int32),
            jax.ShapeDtypeStruct((TOKENS, 1), jnp.int32),
            jax.ShapeDtypeStruct((TOKENS, 128), jnp.float32),
            jax.ShapeDtypeStruct((TOKENS, 128), jnp.float32),
            jax.ShapeDtypeStruct((NBLK, 1), jnp.int32),
        ),
        interpret=interpret,
    )(x, Wg1, bg1, Wg2, bg2, Wg3, bg3, Wd, bd)


def _dispatch(x, pos3, wr1, wr2):
    mesh = plsc.VectorSubcoreMesh(core_axis_name="c", subcore_axis_name="s")

    @functools.partial(
        pl.kernel,
        out_type=(jax.ShapeDtypeStruct((NROWS, D), jnp.float32),
                  jax.ShapeDtypeStruct((NROWS, 128), jnp.float32)),
        mesh=mesh,
        scratch_types=[pltpu.VMEM((TPW, D), jnp.float32),
                       pltpu.VMEM((2, TPW), jnp.int32),
                       pltpu.VMEM((TPW, 128), jnp.float32),
                       pltpu.VMEM((TPW, 128), jnp.float32)],
    )
    def k(x_hbm, pos_hbm, wr1_hbm, wr2_hbm, xs_hbm, ws_hbm, xv, pv, wv1, wv2):
        w = lax.axis_index("s") * 2 + lax.axis_index("c")
        base = w * TPW
        pltpu.sync_copy(pos_hbm.at[w], pv)
        pltpu.sync_copy(x_hbm.at[pl.ds(base, TPW)], xv)
        pltpu.sync_copy(wr1_hbm.at[pl.ds(base, TPW)], wv1)
        pltpu.sync_copy(wr2_hbm.at[pl.ds(base, TPW)], wv2)
        pltpu.sync_copy(xv, xs_hbm.at[pv.at[0]])
        pltpu.sync_copy(xv, xs_hbm.at[pv.at[1]])
        pltpu.sync_copy(wv1, ws_hbm.at[pv.at[0]])
        pltpu.sync_copy(wv2, ws_hbm.at[pv.at[1]])

    return k(x, pos3, wr1, wr2)


def _ffn_body(be_ref, xs_ref, ws_ref, w1_ref, b1_ref, w2_ref, b2_ref,
              os_ref):
    i = pl.program_id(0)

    @pl.when(be_ref[i] >= 0)
    def _():
        xb = xs_ref[...].astype(jnp.bfloat16)
        h = jax.nn.relu(
            jnp.dot(xb, w1_ref[0], preferred_element_type=jnp.float32)
            + b1_ref[0, 0])
        o = jnp.dot(h.astype(jnp.bfloat16), w2_ref[0],
                    preferred_element_type=jnp.float32) + b2_ref[0, 0]
        conf = jax.nn.sigmoid(jnp.mean(o, axis=-1))
        wv = ws_ref[:, 0]
        os_ref[...] = o * (wv * conf)[:, None]


def _ffn(be_flat, xs, ws, W1b, b1r, W2b, b2r, interpret=False):
    grid_spec = pltpu.PrefetchScalarGridSpec(
        num_scalar_prefetch=1,
        grid=(NBLK,),
        in_specs=[
            pl.BlockSpec((TILE, D), lambda i, be: (i, 0)),
            pl.BlockSpec((TILE, 128), lambda i, be: (i, 0)),
            pl.BlockSpec((1, D, D_FF),
                         lambda i, be: (jnp.maximum(be[i], 0), 0, 0)),
            pl.BlockSpec((1, 1, D_FF),
                         lambda i, be: (jnp.maximum(be[i], 0), 0, 0)),
            pl.BlockSpec((1, D_FF, D),
                         lambda i, be: (jnp.maximum(be[i], 0), 0, 0)),
            pl.BlockSpec((1, 1, D),
                         lambda i, be: (jnp.maximum(be[i], 0), 0, 0)),
        ],
        out_specs=pl.BlockSpec((TILE, D), lambda i, be: (i, 0)),
    )
    return pl.pallas_call(
        _ffn_body,
        grid_spec=grid_spec,
        out_shape=jax.ShapeDtypeStruct((NROWS, D), jnp.float32),
        interpret=interpret,
    )(be_flat, xs, ws, W1b, b1r, W2b, b2r)


def _combine(os_rows, pos3):
    mesh = plsc.VectorSubcoreMesh(core_axis_name="c", subcore_axis_name="s")

    @functools.partial(
        pl.kernel,
        out_type=jax.ShapeDtypeStruct((TOKENS, D), jnp.float32),
        mesh=mesh,
        scratch_types=[pltpu.VMEM((TPW, D), jnp.float32),
                       pltpu.VMEM((TPW, D), jnp.float32),
                       pltpu.VMEM((2, TPW), jnp.int32),
                       pltpu.SemaphoreType.DMA,
                       pltpu.SemaphoreType.DMA],
    )
    def k(os_hbm, pos_hbm, out_hbm, g0, g1, pv, sa, sb):
        w = lax.axis_index("s") * 2 + lax.axis_index("c")
        pltpu.sync_copy(pos_hbm.at[w], pv)
        ca = pltpu.async_copy(os_hbm.at[pv.at[0]], g0, sa)
        cb = pltpu.async_copy(os_hbm.at[pv.at[1]], g1, sb)
        ca.wait()
        cb.wait()

        @pl.loop(0, TPW)
        def _(i):
            for c in range(D // 16):
                sl = pl.ds(c * 16, 16)
                g0[i, sl] = g0[i, sl] + g1[i, sl]

        pltpu.sync_copy(g0, out_hbm.at[pl.ds(w * TPW, TPW)])

    return k(os_rows, pos3)


def kernel(x, Wg1, bg1, Wg2, bg2, Wg3, bg3, Wd, bd, W1, b1, W2, b2):
    pos1, pos2, wr1, wr2, be = _gate_route(x, Wg1, bg1, Wg2, bg2, Wg3, bg3,
                                           Wd, bd)
    pos3 = jnp.stack([pos1.reshape(NW, TPW), pos2.reshape(NW, TPW)], axis=1)
    xs, ws = _dispatch(x, pos3, wr1, wr2)
    os_rows = _ffn(be.reshape(NBLK), xs, ws,
                   W1.astype(jnp.bfloat16),
                   b1.reshape(NUM_EXPERTS, 1, D_FF),
                   W2.astype(jnp.bfloat16),
                   b2.reshape(NUM_EXPERTS, 1, D))
    return _combine(os_rows, pos3)


# speedup vs baseline: 1.1049x; 1.1049x over previous
import functools

import jax
import jax.numpy as jnp
from jax import lax
from jax.experimental import pallas as pl
from jax.experimental.pallas import tpu as pltpu
from jax.experimental.pallas import tpu_sc as plsc

NUM_EXPERTS = 8
D = 768
D_FF = 1536
TOKENS = 2048
TILE = 512
SLOTS = 2 * TOKENS
NROWS = SLOTS + NUM_EXPERTS * TILE
NBLK = NROWS // TILE
NW = 32
TPW = TOKENS // NW


def _mm(a, b):
    return jnp.dot(a.astype(jnp.bfloat16), b.astype(jnp.bfloat16),
                   preferred_element_type=jnp.float32)


def _gate_route_body(x_ref, wg1_ref, bg1_ref, wg2_ref, bg2_ref, wg3_ref,
                     bg3_ref, wd_ref, bd_ref,
                     pos1_ref, pos2_ref, wr1_ref, wr2_ref, be_ref):
    x = x_ref[...]
    g = jax.nn.relu(_mm(x, wg1_ref[...]) + bg1_ref[...])
    g = jax.nn.relu(_mm(g, wg2_ref[...]) + bg2_ref[...])
    logits = _mm(g, wg3_ref[...]) + bg3_ref[...]
    logits = logits + (_mm(x, wd_ref[...]) + bd_ref[...]) * 0.1
    p = jax.nn.softmax(logits, axis=-1)

    lane = lax.broadcasted_iota(jnp.int32, (TOKENS, NUM_EXPERTS), 1)
    m1 = jnp.max(p, axis=-1, keepdims=True)
    i1 = jnp.min(jnp.where(p == m1, lane, NUM_EXPERTS), axis=-1, keepdims=True)
    pm = jnp.where(lane == i1, -jnp.inf, p)
    m2 = jnp.max(pm, axis=-1, keepdims=True)
    i2 = jnp.min(jnp.where(pm == m2, lane, NUM_EXPERTS), axis=-1, keepdims=True)
    e2 = jnp.exp(m2 - m1)
    s1 = 1.0 / (1.0 + e2)
    s2 = e2 / (1.0 + e2)
    s1 = jnp.where(s1 > 0.01, s1, 0.0)
    s2 = jnp.where(s2 > 0.01, s2, 0.0)

    oh1 = (lane == i1).astype(jnp.bfloat16)
    oh2 = (lane == i2).astype(jnp.bfloat16)
    oh = jnp.concatenate([oh1, oh2], axis=1)
    r256 = lax.broadcasted_iota(jnp.int32, (256, 256), 0)
    c256 = lax.broadcasted_iota(jnp.int32, (256, 256), 1)
    tri = (r256 > c256).astype(jnp.bfloat16)
    carry = jnp.zeros((1, 2 * NUM_EXPERTS), jnp.float32)
    parts = []
    for b in range(TOKENS // 256):
        blk = lax.slice(oh, (b * 256, 0), ((b + 1) * 256, 2 * NUM_EXPERTS))
        parts.append(jnp.dot(tri, blk, preferred_element_type=jnp.float32)
                     + carry)
        carry = carry + jnp.sum(blk.astype(jnp.float32), axis=0,
                                keepdims=True)
    pfx = jnp.concatenate(parts, axis=0)
    p1 = pfx[:, :NUM_EXPERTS]
    p2 = pfx[:, NUM_EXPERTS:]
    c1 = carry[:, :NUM_EXPERTS]
    c2 = carry[:, NUM_EXPERTS:]
    counts = (c1 + c2).astype(jnp.int32)
    padded = ((counts + (TILE - 1)) // TILE) * TILE
    r8 = lax.broadcasted_iota(jnp.int32, (NUM_EXPERTS, NUM_EXPERTS), 0)
    c8 = lax.broadcasted_iota(jnp.int32, (NUM_EXPERTS, NUM_EXPERTS), 1)
    triu8 = (r8 < c8).astype(jnp.bfloat16)
    off = jnp.dot(padded.astype(jnp.bfloat16), triu8,
                  preferred_element_type=jnp.float32)

    rank1 = jnp.sum(jnp.where(lane == i1, p1, 0.0), axis=1, keepdims=True)
    rank2 = jnp.sum(jnp.where(lane == i2, p2 + c1, 0.0), axis=1,
                    keepdims=True)
    off1 = jnp.sum(jnp.where(lane == i1, off, 0.0), axis=1, keepdims=True)
    off2 = jnp.sum(jnp.where(lane == i2, off, 0.0), axis=1, keepdims=True)
    pos1_ref[...] = (off1 + rank1).astype(jnp.int32)
    pos2_ref[...] = (off2 + rank2).astype(jnp.int32)

    ones128 = jnp.ones((1, 128), jnp.float32)
    wr1_ref[...] = s1 * ones128
    wr2_ref[...] = s2 * ones128

    total = jnp.sum(padded, axis=1, keepdims=True)
    bid = lax.broadcasted_iota(jnp.int32, (NBLK, 1), 0) * TILE
    ge = (bid >= off.astype(jnp.int32)).astype(jnp.int32)
    be = jnp.sum(ge, axis=1, keepdims=True) - 1
    be_ref[...] = jnp.where(bid < total, be, -1)


def _gate_route(x, Wg1, bg1, Wg2, bg2, Wg3, bg3, Wd, bd, interpret=False):
    return pl.pallas_call(
        _gate_route_body,
        out_shape=(
            jax.ShapeDtypeStruct((TOKENS, 1), jnp.int32),
            jax.ShapeDtypeStruct((TOKENS, 1), jnp.int32),
            jax.ShapeDtypeStruct((TOKENS, 128), jnp.float32),
            jax.ShapeDtypeStruct((TOKENS, 128), jnp.float32),
            jax.ShapeDtypeStruct((NBLK, 1), jnp.int32),
        ),
        interpret=interpret,
    )(x, Wg1, bg1, Wg2, bg2, Wg3, bg3, Wd, bd)


def _dispatch(x, pos3, wr1, wr2):
    mesh = plsc.VectorSubcoreMesh(core_axis_name="c", subcore_axis_name="s")

    @functools.partial(
        pl.kernel,
        out_type=(jax.ShapeDtypeStruct((NROWS, D), jnp.float32),
                  jax.ShapeDtypeStruct((NROWS, 128), jnp.float32)),
        mesh=mesh,
        scratch_types=[pltpu.VMEM((TPW, D), jnp.float32),
                       pltpu.VMEM((2, TPW), jnp.int32),
                       pltpu.VMEM((TPW, 128), jnp.float32),
                       pltpu.VMEM((TPW, 128), jnp.float32)],
    )
    def k(x_hbm, pos_hbm, wr1_hbm, wr2_hbm, xs_hbm, ws_hbm, xv, pv, wv1, wv2):
        w = lax.axis_index("s") * 2 + lax.axis_index("c")
        base = w * TPW
        pltpu.sync_copy(pos_hbm.at[w], pv)
        pltpu.sync_copy(x_hbm.at[pl.ds(base, TPW)], xv)
        pltpu.sync_copy(wr1_hbm.at[pl.ds(base, TPW)], wv1)
        pltpu.sync_copy(wr2_hbm.at[pl.ds(base, TPW)], wv2)
        pltpu.sync_copy(xv, xs_hbm.at[pv.at[0]])
        pltpu.sync_copy(xv, xs_hbm.at[pv.at[1]])
        pltpu.sync_copy(wv1, ws_hbm.at[pv.at[0]])
        pltpu.sync_copy(wv2, ws_hbm.at[pv.at[1]])

    return k(x, pos3, wr1, wr2)


def _ffn_body(be_ref, xs_ref, ws_ref, w1_ref, b1_ref, w2_ref, b2_ref,
              os_ref):
    i = pl.program_id(0)

    @pl.when(be_ref[i] >= 0)
    def _():
        xb = xs_ref[...].astype(jnp.bfloat16)
        h = jax.nn.relu(
            jnp.dot(xb, w1_ref[0], preferred_element_type=jnp.float32)
            + b1_ref[0, 0])
        o = jnp.dot(h.astype(jnp.bfloat16), w2_ref[0],
                    preferred_element_type=jnp.float32) + b2_ref[0, 0]
        conf = jax.nn.sigmoid(jnp.mean(o, axis=-1))
        wv = ws_ref[:, 0]
        os_ref[...] = o * (wv * conf)[:, None]


def _ffn(be_flat, xs, ws, W1b, b1r, W2b, b2r, interpret=False):
    grid_spec = pltpu.PrefetchScalarGridSpec(
        num_scalar_prefetch=1,
        grid=(NBLK,),
        in_specs=[
            pl.BlockSpec((TILE, D), lambda i, be: (i, 0)),
            pl.BlockSpec((TILE, 128), lambda i, be: (i, 0)),
            pl.BlockSpec((1, D, D_FF),
                         lambda i, be: (jnp.maximum(be[i], 0), 0, 0)),
            pl.BlockSpec((1, 1, D_FF),
                         lambda i, be: (jnp.maximum(be[i], 0), 0, 0)),
            pl.BlockSpec((1, D_FF, D),
                         lambda i, be: (jnp.maximum(be[i], 0), 0, 0)),
            pl.BlockSpec((1, 1, D),
                         lambda i, be: (jnp.maximum(be[i], 0), 0, 0)),
        ],
        out_specs=pl.BlockSpec((TILE, D), lambda i, be: (i, 0)),
    )
    return pl.pallas_call(
        _ffn_body,
        grid_spec=grid_spec,
        out_shape=jax.ShapeDtypeStruct((NROWS, D), jnp.float32),
        interpret=interpret,
    )(be_flat, xs, ws, W1b, b1r, W2b, b2r)


def _combine(os_rows, pos3):
    mesh = plsc.VectorSubcoreMesh(core_axis_name="c", subcore_axis_name="s")

    @functools.partial(
        pl.kernel,
        out_type=jax.ShapeDtypeStruct((TOKENS, D), jnp.float32),
        mesh=mesh,
        scratch_types=[pltpu.VMEM((TPW, D), jnp.float32),
                       pltpu.VMEM((TPW, D), jnp.float32),
                       pltpu.VMEM((2, TPW), jnp.int32),
                       pltpu.SemaphoreType.DMA,
                       pltpu.SemaphoreType.DMA],
    )
    def k(os_hbm, pos_hbm, out_hbm, g0, g1, pv, sa, sb):
        w = lax.axis_index("s") * 2 + lax.axis_index("c")
        pltpu.sync_copy(pos_hbm.at[w], pv)
        ca = pltpu.async_copy(os_hbm.at[pv.at[0]], g0, sa)
        cb = pltpu.async_copy(os_hbm.at[pv.at[1]], g1, sb)
        ca.wait()
        cb.wait()

        @pl.loop(0, TPW)
        def _(i):
            for c in range(D // 16):
                sl = pl.ds(c * 16, 16)
                g0[i, sl] = g0[i, sl] + g1[i, sl]

        pltpu.sync_copy(g0, out_hbm.at[pl.ds(w * TPW, TPW)])

    return k(os_rows, pos3)


def kernel(x, Wg1, bg1, Wg2, bg2, Wg3, bg3, Wd, bd, W1, b1, W2, b2):
    pos1, pos2, wr1, wr2, be = _gate_route(x, Wg1, bg1, Wg2, bg2, Wg3, bg3,
                                           Wd, bd)
    pos3 = jnp.stack([pos1.reshape(NW, TPW), pos2.reshape(NW, TPW)], axis=1)
    xs, ws = _dispatch(x, pos3, wr1, wr2)
    os_rows = _ffn(be.reshape(NBLK), xs, ws,
                   W1.astype(jnp.bfloat16),
                   b1.reshape(NUM_EXPERTS, 1, D_FF),
                   W2.astype(jnp.bfloat16),
                   b2.reshape(NUM_EXPERTS, 1, D))
    return _combine(os_rows, pos3)
